# Initial kernel scaffold; baseline (speedup 1.0000x reference)
import jax
import jax.numpy as jnp
from jax.experimental import pallas as pl


def kernel(x, edge_index, edge_attr, We, be, W1, b1, W2, b2):
    raise NotImplementedError("write your pallas kernel here")


# speedup vs baseline: 9.6059x; 9.6059x over previous
import functools

import jax
import jax.numpy as jnp
from jax import lax
from jax.experimental import pallas as pl
from jax.experimental.pallas import tpu as pltpu
from jax.experimental.pallas import tpu_sc as plsc

N = 10000
E = 320000
D = 128

NC, NS = 2, 16
NW = NC * NS
NPAD = 10240
EPW = E // NW
CH = 80
NCHUNK = EPW // CH
NODES_PW = NPAD // NW
SELF_CHUNKS = NODES_PW // CH
SLICE_PS = NPAD // NS

_MESH = plsc.VectorSubcoreMesh(core_axis_name="c", subcore_axis_name="s")


def _deg_body(dst_hbm, ew_hbm, out_hbm, idx_v, ew_v, z_v, deg_sh):
    c = lax.axis_index("c")
    s = lax.axis_index("s")
    w = c * NS + s

    def zfill(i, _):
        z_v[pl.ds(i * 16, 16)] = jnp.zeros((16,), jnp.float32)
        return 0

    lax.fori_loop(0, SLICE_PS // 16, zfill, 0)
    pltpu.sync_copy(z_v, deg_sh.at[pl.ds(s * SLICE_PS, SLICE_PS)])
    plsc.subcore_barrier()

    base = w * EPW

    def chunk(j, _):
        off = base + j * CH
        pltpu.sync_copy(dst_hbm.at[pl.ds(off, CH)], idx_v.at[0])
        pltpu.sync_copy(ew_hbm.at[pl.ds(off, CH)], ew_v)
        pltpu.sync_copy(ew_v, deg_sh.at[idx_v.at[0]], add=True)
        return 0

    lax.fori_loop(0, NCHUNK, chunk, 0)
    plsc.subcore_barrier()
    pltpu.sync_copy(deg_sh.at[pl.ds(s * SLICE_PS, SLICE_PS)],
                    out_hbm.at[c, pl.ds(s * SLICE_PS, SLICE_PS)])


_deg_call = functools.partial(
    pl.kernel,
    out_type=jax.ShapeDtypeStruct((NC, NPAD), jnp.float32),
    mesh=_MESH,
    scratch_types=[
        pltpu.VMEM((1, CH), jnp.int32),
        pltpu.VMEM((CH,), jnp.float32),
        pltpu.VMEM((SLICE_PS,), jnp.float32),
        pltpu.VMEM_SHARED((NPAD,), jnp.float32),
    ],
)(_deg_body)


def _spmm_body(xw_hbm, src_hbm, dst_hbm, ew_hbm, out_hbm,
               srcv, dstv, ewv, rows, selfv, acc_sh, gsem):
    c = lax.axis_index("c")
    s = lax.axis_index("s")
    w = c * NS + s

    def zfill(i, _):
        rows[i // 8, pl.ds((i % 8) * 16, 16)] = jnp.zeros((16,), jnp.float32)
        return 0

    lax.fori_loop(0, CH * 8, zfill, 0)

    def zcopy(k, _):
        pltpu.sync_copy(rows, acc_sh.at[pl.ds(s * SLICE_PS + k * CH, CH)])
        return 0

    lax.fori_loop(0, SLICE_PS // CH, zcopy, 0)
    plsc.subcore_barrier()

    base = w * EPW

    def chunk(j, _):
        off = base + j * CH
        pltpu.sync_copy(src_hbm.at[pl.ds(off, CH)], srcv.at[0])
        pltpu.sync_copy(dst_hbm.at[pl.ds(off, CH)], dstv.at[0])
        pltpu.sync_copy(ew_hbm.at[pl.ds(off, CH)], ewv)
        pltpu.async_copy(xw_hbm.at[srcv.at[0]], rows, gsem).wait()

        def scale(g, _):
            evec = ewv[pl.ds(g * 16, 16)]
            for l in range(16):
                r = g * 16 + l
                v = jnp.broadcast_to(evec[l], (16,))
                for cc in range(8):
                    rows[r, pl.ds(cc * 16, 16)] = (
                        rows[r, pl.ds(cc * 16, 16)] * v)
            return 0

        lax.fori_loop(0, CH // 16, scale, 0)
        pltpu.sync_copy(rows, acc_sh.at[dstv.at[0]], add=True)
        return 0

    lax.fori_loop(0, NCHUNK, chunk, 0)

    nbase = w * NODES_PW

    def schunk(j2, _):
        coff = nbase + j2 * CH

        @pl.when(coff < N)
        def _():
            def sfill(i, _):
                selfv[0, pl.ds(i * 16, 16)] = (
                    lax.iota(jnp.int32, 16) + (coff + i * 16))
                return 0

            lax.fori_loop(0, CH // 16, sfill, 0)
            pltpu.sync_copy(xw_hbm.at[pl.ds(coff, CH)], rows)
            pltpu.sync_copy(rows, acc_sh.at[selfv.at[0]], add=True)

        return 0

    lax.fori_loop(0, SELF_CHUNKS, schunk, 0)
    plsc.subcore_barrier()
    pltpu.sync_copy(acc_sh.at[pl.ds(s * SLICE_PS, SLICE_PS)],
                    out_hbm.at[c, pl.ds(s * SLICE_PS, SLICE_PS)])


_spmm_call = functools.partial(
    pl.kernel,
    out_type=jax.ShapeDtypeStruct((NC, NPAD, D), jnp.float32),
    mesh=_MESH,
    scratch_types=[
        pltpu.VMEM((1, CH), jnp.int32),
        pltpu.VMEM((1, CH), jnp.int32),
        pltpu.VMEM((CH,), jnp.float32),
        pltpu.VMEM((CH, D), jnp.float32),
        pltpu.VMEM((1, CH), jnp.int32),
        pltpu.VMEM_SHARED((NPAD, D), jnp.float32),
        pltpu.SemaphoreType.DMA,
    ],
)(_spmm_body)


def _dinv_body(parts_ref, dinv_ref):
    deg = parts_ref[0] + parts_ref[1] + 1.0
    dinv_ref[...] = jnp.where(deg > 0, lax.rsqrt(deg), 0.0)


def _enc_body(x_ref, we_ref, be_ref, w1_ref, dv_ref, o_ref):
    h = jnp.tanh(
        jnp.dot(x_ref[...], we_ref[...], preferred_element_type=jnp.float32)
        + be_ref[...])
    o_ref[...] = jnp.dot(
        h, w1_ref[...], preferred_element_type=jnp.float32) * dv_ref[...]


def _mid_body(p0_ref, p1_ref, b1_ref, w2_ref, dv_ref, o_ref):
    h = jnp.tanh((p0_ref[...] + p1_ref[...]) * dv_ref[...] + b1_ref[...])
    o_ref[...] = jnp.dot(
        h, w2_ref[...], preferred_element_type=jnp.float32) * dv_ref[...]


def _out_body(p0_ref, p1_ref, b2_ref, dv_ref, o_ref):
    o_ref[...] = jnp.tanh(
        (p0_ref[...] + p1_ref[...]) * dv_ref[...] + b2_ref[...])


_RB = 1000
_GRID = N // _RB


def kernel(x, edge_index, edge_attr, We, be, W1, b1, W2, b2):
    src = edge_index[0].astype(jnp.int32)
    dst = edge_index[1].astype(jnp.int32)
    ew = edge_attr.astype(jnp.float32)
    be2 = be.reshape(1, D)
    b12 = b1.reshape(1, D)
    b22 = b2.reshape(1, D)

    deg_parts = _deg_call(dst, ew)

    dinv2d = pl.pallas_call(
        _dinv_body,
        out_shape=jax.ShapeDtypeStruct((NPAD // 128, 128), jnp.float32),
    )(deg_parts.reshape(NC, NPAD // 128, 128))
    dinv_col = dinv2d.reshape(NPAD, 1)[:N]

    blk_row = pl.BlockSpec((_RB, D), lambda i: (i, 0))
    blk_mat = pl.BlockSpec((D, D), lambda i: (0, 0))
    blk_vec = pl.BlockSpec((1, D), lambda i: (0, 0))
    blk_col = pl.BlockSpec((_RB, 1), lambda i: (i, 0))
    out_row = jax.ShapeDtypeStruct((N, D), jnp.float32)

    xw1 = pl.pallas_call(
        _enc_body,
        grid=(_GRID,),
        in_specs=[blk_row, blk_mat, blk_vec, blk_mat, blk_col],
        out_specs=blk_row,
        out_shape=out_row,
    )(x, We, be2, W1, dinv_col)

    parts1 = _spmm_call(xw1, src, dst, ew)

    xw2 = pl.pallas_call(
        _mid_body,
        grid=(_GRID,),
        in_specs=[blk_row, blk_row, blk_vec, blk_mat, blk_col],
        out_specs=blk_row,
        out_shape=out_row,
    )(parts1[0], parts1[1], b12, W2, dinv_col)

    parts2 = _spmm_call(xw2, src, dst, ew)

    out = pl.pallas_call(
        _out_body,
        grid=(_GRID,),
        in_specs=[blk_row, blk_row, blk_vec, blk_col],
        out_specs=blk_row,
        out_shape=out_row,
    )(parts2[0], parts2[1], b22, dinv_col)
    return out


# speedup vs baseline: 10.8720x; 1.1318x over previous
import functools

import jax
import jax.numpy as jnp
from jax import lax
from jax.experimental import pallas as pl
from jax.experimental.pallas import tpu as pltpu
from jax.experimental.pallas import tpu_sc as plsc

N = 10000
E = 320000
D = 128

NC, NS = 2, 16
NW = NC * NS
NPAD = 10240
CH = 48
CPW = 216
EEXT = NW * CPW * CH
EBN = 7
SLICE_PS = NPAD // NS

_MESH = plsc.VectorSubcoreMesh(core_axis_name="c", subcore_axis_name="s")


def _deg_body(dst_hbm, ew_hbm, out_hbm, dstall, ewall, z_v, deg_sh):
    c = lax.axis_index("c")
    s = lax.axis_index("s")
    w = c * NS + s

    def zfill(i, _):
        z_v[pl.ds(i * 16, 16)] = jnp.zeros((16,), jnp.float32)
        return 0

    lax.fori_loop(0, SLICE_PS // 16, zfill, 0)
    pltpu.sync_copy(z_v, deg_sh.at[pl.ds(s * SLICE_PS, SLICE_PS)])

    pltpu.sync_copy(dst_hbm.at[w], dstall)
    pltpu.sync_copy(ew_hbm.at[w], ewall)
    plsc.subcore_barrier()

    def fire(j, _):
        pltpu.sync_copy(ewall.at[j], deg_sh.at[dstall.at[j]], add=True)
        return 0

    lax.fori_loop(0, CPW, fire, 0)
    plsc.subcore_barrier()
    pltpu.sync_copy(deg_sh.at[pl.ds(s * SLICE_PS, SLICE_PS)],
                    out_hbm.at[c, pl.ds(s * SLICE_PS, SLICE_PS)])


_deg_call = functools.partial(
    pl.kernel,
    out_type=jax.ShapeDtypeStruct((NC, NPAD), jnp.float32),
    mesh=_MESH,
    scratch_types=[
        pltpu.VMEM((CPW, CH), jnp.int32),
        pltpu.VMEM((CPW, CH), jnp.float32),
        pltpu.VMEM((SLICE_PS,), jnp.float32),
        pltpu.VMEM_SHARED((NPAD,), jnp.float32),
    ],
)(_deg_body)


def _spmm_body(xw_hbm, ed_hbm, ew_hbm, out_hbm,
               ebuf, ewbuf, rows, acc_sh,
               gsem0, gsem1, ssem0, ssem1, isem0, isem1):
    c = lax.axis_index("c")
    s = lax.axis_index("s")
    w = c * NS + s

    def zfill(i, _):
        rows[0, i // 8, pl.ds((i % 8) * 16, 16)] = jnp.zeros((16,), jnp.float32)
        return 0

    lax.fori_loop(0, CH * 8, zfill, 0)

    def zcopy(k, _):
        pltpu.sync_copy(rows.at[0], acc_sh.at[pl.ds(s * SLICE_PS + k * CH, CH)])
        return 0

    lax.fori_loop(0, SLICE_PS // CH, zcopy, 0)
    pltpu.sync_copy(
        rows.at[0, pl.ds(0, SLICE_PS - (SLICE_PS // CH) * CH)],
        acc_sh.at[pl.ds(s * SLICE_PS + (SLICE_PS // CH) * CH,
                        SLICE_PS - (SLICE_PS // CH) * CH)])
    plsc.subcore_barrier()

    gsems = (gsem0, gsem1)
    ssems = (ssem0, ssem1)
    isems = (isem0, isem1)

    def istart(j, p):
        jm = lax.rem(j, EBN)
        pltpu.async_copy(ed_hbm.at[w, j], ebuf.at[jm], isems[p])
        pltpu.async_copy(ew_hbm.at[w, j], ewbuf.at[jm], isems[p])

    def iwait(p):
        pltpu.make_async_copy(ed_hbm.at[0, 0], ebuf.at[0], isems[p]).wait()
        pltpu.make_async_copy(ew_hbm.at[0, 0], ewbuf.at[0], isems[p]).wait()

    def gstart(j, p):
        pltpu.async_copy(xw_hbm.at[ebuf.at[lax.rem(j, EBN), 0]],
                         rows.at[lax.rem(j, 4)], gsems[p])

    def gwait(p):
        pltpu.make_async_copy(xw_hbm.at[pl.ds(0, CH)], rows.at[0],
                              gsems[p]).wait()

    def sstart(j, p):
        pltpu.async_copy(rows.at[lax.rem(j, 4)],
                         acc_sh.at[ebuf.at[lax.rem(j, EBN), 1]],
                         ssems[p], add=True)

    def swait(p):
        pltpu.make_async_copy(xw_hbm.at[pl.ds(0, CH)], rows.at[0],
                              ssems[p]).wait()

    def scale(j):
        jm = lax.rem(j, EBN)
        bm = lax.rem(j, 4)

        def group(g, _):
            evec = ewbuf[jm, pl.ds(g * 16, 16)]
            for l in range(16):
                v = jnp.broadcast_to(evec[l], (16,))
                r = g * 16 + l
                for cc in range(8):
                    rows[bm, r, pl.ds(cc * 16, 16)] = (
                        rows[bm, r, pl.ds(cc * 16, 16)] * v)
            return 0

        lax.fori_loop(0, CH // 16, group, 0)

    istart(0, 0)
    istart(1, 1)
    iwait(0)
    gstart(0, 0)
    istart(2, 0)
    iwait(1)
    gstart(1, 1)
    istart(3, 1)
    for j0 in (0, 1):
        p = j0 % 2
        gwait(p)
        iwait(p)
        gstart(j0 + 2, p)
        istart(j0 + 4, p)
        scale(j0)
        sstart(j0, p)

    def steady(j, p):
        gwait(p)
        swait(p)

        @pl.when(j + 2 < CPW)
        def _():
            iwait(p)
            gstart(j + 2, p)

        @pl.when(j + 4 < CPW)
        def _():
            istart(j + 4, p)

        scale(j)
        sstart(j, p)

    def pair(i, _):
        steady(2 + 2 * i, 0)
        steady(3 + 2 * i, 1)
        return 0

    lax.fori_loop(0, (CPW - 2) // 2, pair, 0)
    swait(0)
    swait(1)
    plsc.subcore_barrier()
    pltpu.sync_copy(acc_sh.at[pl.ds(s * SLICE_PS, SLICE_PS)],
                    out_hbm.at[c, pl.ds(s * SLICE_PS, SLICE_PS)])


_spmm_call = functools.partial(
    pl.kernel,
    out_type=jax.ShapeDtypeStruct((NC, NPAD, D), jnp.float32),
    mesh=_MESH,
    scratch_types=[
        pltpu.VMEM((EBN, 2, CH), jnp.int32),
        pltpu.VMEM((EBN, CH), jnp.float32),
        pltpu.VMEM((4, CH, D), jnp.float32),
        pltpu.VMEM_SHARED((NPAD, D), jnp.float32),
        pltpu.SemaphoreType.DMA,
        pltpu.SemaphoreType.DMA,
        pltpu.SemaphoreType.DMA,
        pltpu.SemaphoreType.DMA,
        pltpu.SemaphoreType.DMA,
        pltpu.SemaphoreType.DMA,
    ],
)(_spmm_body)


def _dinv_body(parts_ref, dinv_ref):
    deg = parts_ref[0] + parts_ref[1]
    dinv_ref[...] = jnp.where(deg > 0, lax.rsqrt(deg), 0.0)


def _enc_body(x_ref, we_ref, be_ref, w1_ref, dv_ref, o_ref):
    h = jnp.tanh(
        jnp.dot(x_ref[...], we_ref[...], preferred_element_type=jnp.float32)
        + be_ref[...])
    o_ref[...] = jnp.dot(
        h, w1_ref[...], preferred_element_type=jnp.float32) * dv_ref[...]


def _mid_body(p0_ref, p1_ref, b1_ref, w2_ref, dv_ref, o_ref):
    h = jnp.tanh((p0_ref[...] + p1_ref[...]) * dv_ref[...] + b1_ref[...])
    o_ref[...] = jnp.dot(
        h, w2_ref[...], preferred_element_type=jnp.float32) * dv_ref[...]


def _out_body(p0_ref, p1_ref, b2_ref, dv_ref, o_ref):
    o_ref[...] = jnp.tanh(
        (p0_ref[...] + p1_ref[...]) * dv_ref[...] + b2_ref[...])


_RB = 1000
_GRID = N // _RB


def kernel(x, edge_index, edge_attr, We, be, W1, b1, W2, b2):
    src = edge_index[0].astype(jnp.int32)
    dst = edge_index[1].astype(jnp.int32)
    ew = edge_attr.astype(jnp.float32)
    nodes = jnp.arange(NPAD, dtype=jnp.int32)
    pad_i = jnp.arange(EEXT - E - NPAD, dtype=jnp.int32)
    src_ext = jnp.concatenate(
        [src, jnp.minimum(nodes, N - 1), pad_i % N])
    dst_ext = jnp.concatenate([dst, nodes, N + pad_i % (NPAD - N)])
    ew_ext = jnp.concatenate(
        [ew, (nodes < N).astype(jnp.float32),
         jnp.zeros(EEXT - E - NPAD, jnp.float32)])
    dst2d = dst_ext.reshape(NW, CPW, CH)
    ew2d = ew_ext.reshape(NW, CPW, CH)
    ed = jnp.concatenate([
        src_ext.reshape(NW, CPW, 1, CH),
        dst_ext.reshape(NW, CPW, 1, CH),
    ], axis=2)
    be2 = be.reshape(1, D)
    b12 = b1.reshape(1, D)
    b22 = b2.reshape(1, D)

    deg_parts = _deg_call(dst2d, ew2d)

    dinv2d = pl.pallas_call(
        _dinv_body,
        out_shape=jax.ShapeDtypeStruct((NPAD // 128, 128), jnp.float32),
    )(deg_parts.reshape(NC, NPAD // 128, 128))
    dinv_col = dinv2d.reshape(NPAD, 1)[:N]

    blk_row = pl.BlockSpec((_RB, D), lambda i: (i, 0))
    blk_mat = pl.BlockSpec((D, D), lambda i: (0, 0))
    blk_vec = pl.BlockSpec((1, D), lambda i: (0, 0))
    blk_col = pl.BlockSpec((_RB, 1), lambda i: (i, 0))
    out_row = jax.ShapeDtypeStruct((N, D), jnp.float32)

    xw1 = pl.pallas_call(
        _enc_body,
        grid=(_GRID,),
        in_specs=[blk_row, blk_mat, blk_vec, blk_mat, blk_col],
        out_specs=blk_row,
        out_shape=out_row,
    )(x, We, be2, W1, dinv_col)

    parts1 = _spmm_call(xw1, ed, ew2d)

    xw2 = pl.pallas_call(
        _mid_body,
        grid=(_GRID,),
        in_specs=[blk_row, blk_row, blk_vec, blk_mat, blk_col],
        out_specs=blk_row,
        out_shape=out_row,
    )(parts1[0], parts1[1], b12, W2, dinv_col)

    parts2 = _spmm_call(xw2, ed, ew2d)

    out = pl.pallas_call(
        _out_body,
        grid=(_GRID,),
        in_specs=[blk_row, blk_row, blk_vec, blk_col],
        out_specs=blk_row,
        out_shape=out_row,
    )(parts2[0], parts2[1], b22, dinv_col)
    return out


# speedup vs baseline: 24.2387x; 2.2295x over previous
import functools

import jax
import jax.numpy as jnp
from jax import lax
from jax.experimental import pallas as pl
from jax.experimental.pallas import tpu as pltpu
from jax.experimental.pallas import tpu_sc as plsc

N = 10000
E = 320000
D = 128

NC, NS = 2, 16
NW = NC * NS
NPAD = 10240
CH = 96
CPW = 108
EEXT = NW * CPW * CH
SLICE_PS = NPAD // NS

_MESH = plsc.VectorSubcoreMesh(core_axis_name="c", subcore_axis_name="s")


def _deg_body(dst_hbm, ew_hbm, out_hbm, dstall, ewall, z_v, deg_sh):
    c = lax.axis_index("c")
    s = lax.axis_index("s")
    w = c * NS + s

    def zfill(i, _):
        z_v[pl.ds(i * 16, 16)] = jnp.zeros((16,), jnp.float32)
        return 0

    lax.fori_loop(0, SLICE_PS // 16, zfill, 0)
    pltpu.sync_copy(z_v, deg_sh.at[pl.ds(s * SLICE_PS, SLICE_PS)])

    pltpu.sync_copy(dst_hbm.at[w], dstall)
    pltpu.sync_copy(ew_hbm.at[w], ewall)
    plsc.subcore_barrier()

    def fire(j, _):
        pltpu.sync_copy(ewall.at[j], deg_sh.at[dstall.at[j]], add=True)
        return 0

    lax.fori_loop(0, CPW, fire, 0)
    plsc.subcore_barrier()
    pltpu.sync_copy(deg_sh.at[pl.ds(s * SLICE_PS, SLICE_PS)],
                    out_hbm.at[c, pl.ds(s * SLICE_PS, SLICE_PS)])


_deg_call = functools.partial(
    pl.kernel,
    out_type=jax.ShapeDtypeStruct((NC, NPAD), jnp.float32),
    mesh=_MESH,
    scratch_types=[
        pltpu.VMEM((CPW, CH), jnp.int32),
        pltpu.VMEM((CPW, CH), jnp.float32),
        pltpu.VMEM((SLICE_PS,), jnp.float32),
        pltpu.VMEM_SHARED((NPAD,), jnp.float32),
    ],
)(_deg_body)


def _spmm_body(xw_hbm, ed_hbm, ew_hbm, out_hbm,
               ebuf, ewbuf, rows, acc_sh, gsem, ssem, isem0, isem1):
    c = lax.axis_index("c")
    s = lax.axis_index("s")
    w = c * NS + s

    def zfill(i, _):
        rows[0, i // 8, pl.ds((i % 8) * 16, 16)] = jnp.zeros((16,), jnp.float32)
        return 0

    lax.fori_loop(0, CH * 8, zfill, 0)

    def zcopy(k, _):
        pltpu.sync_copy(rows.at[0], acc_sh.at[pl.ds(s * SLICE_PS + k * CH, CH)])
        return 0

    lax.fori_loop(0, SLICE_PS // CH, zcopy, 0)
    pltpu.sync_copy(
        rows.at[0, pl.ds(0, SLICE_PS - (SLICE_PS // CH) * CH)],
        acc_sh.at[pl.ds(s * SLICE_PS + (SLICE_PS // CH) * CH,
                        SLICE_PS - (SLICE_PS // CH) * CH)])
    plsc.subcore_barrier()

    isems = (isem0, isem1)

    def istart(j, p):
        pltpu.async_copy(ed_hbm.at[w, j], ebuf.at[j % 3], isems[p])
        pltpu.async_copy(ew_hbm.at[w, j], ewbuf.at[j % 3], isems[p])

    def iwait(p):
        pltpu.make_async_copy(ed_hbm.at[0, 0], ebuf.at[0], isems[p]).wait()
        pltpu.make_async_copy(ew_hbm.at[0, 0], ewbuf.at[0], isems[p]).wait()

    def gstart(j, b):
        pltpu.async_copy(xw_hbm.at[ebuf.at[j % 3, 0]], rows.at[b], gsem)

    def gwait(b):
        pltpu.make_async_copy(xw_hbm.at[pl.ds(0, CH)], rows.at[b], gsem).wait()

    def sstart(j, b):
        pltpu.async_copy(rows.at[b], acc_sh.at[ebuf.at[j % 3, 1]], ssem,
                         add=True)

    def swait(b):
        pltpu.make_async_copy(xw_hbm.at[pl.ds(0, CH)], rows.at[b], ssem).wait()

    def scale(j, b):
        def group(g, _):
            evec = ewbuf[j % 3, pl.ds(g * 16, 16)]
            for l in range(16):
                v = jnp.broadcast_to(evec[l], (16,))
                r = g * 16 + l
                for cc in range(8):
                    rows[b, r, pl.ds(cc * 16, 16)] = (
                        rows[b, r, pl.ds(cc * 16, 16)] * v)
            return 0

        lax.fori_loop(0, CH // 16, group, 0)

    def process(j, b, jpar, first, last_possible):
        gwait(b)
        if not first:
            swait(1 - b)
        if last_possible:
            @pl.when(j + 2 < CPW)
            def _():
                istart(j + 2, jpar)

            @pl.when(j + 1 < CPW)
            def _():
                iwait(1 - jpar)
                gstart(j + 1, 1 - b)
        else:
            istart(j + 2, jpar)
            iwait(1 - jpar)
            gstart(j + 1, 1 - b)
        scale(j, b)
        sstart(j, b)

    istart(0, 0)
    istart(1, 1)
    iwait(0)
    gstart(0, 0)
    process(0, 0, 0, first=True, last_possible=False)

    def pair(i, _):
        process(1 + 2 * i, 1, 1, first=False, last_possible=True)
        process(2 + 2 * i, 0, 0, first=False, last_possible=True)
        return 0

    lax.fori_loop(0, (CPW - 1) // 2, pair, 0)
    process(CPW - 1, 1, 1, first=False, last_possible=True)
    swait(1)
    plsc.subcore_barrier()
    pltpu.sync_copy(acc_sh.at[pl.ds(s * SLICE_PS, SLICE_PS)],
                    out_hbm.at[c, pl.ds(s * SLICE_PS, SLICE_PS)])


_spmm_call = functools.partial(
    pl.kernel,
    out_type=jax.ShapeDtypeStruct((NC, NPAD, D), jnp.float32),
    mesh=_MESH,
    scratch_types=[
        pltpu.VMEM((3, 2, CH), jnp.int32),
        pltpu.VMEM((3, CH), jnp.float32),
        pltpu.VMEM((2, CH, D), jnp.float32),
        pltpu.VMEM_SHARED((NPAD, D), jnp.float32),
        pltpu.SemaphoreType.DMA,
        pltpu.SemaphoreType.DMA,
        pltpu.SemaphoreType.DMA,
        pltpu.SemaphoreType.DMA,
    ],
)(_spmm_body)


def _dinv_body(parts_ref, dinv_ref):
    deg = parts_ref[0] + parts_ref[1]
    dinv_ref[...] = jnp.where(deg > 0, lax.rsqrt(deg), 0.0)


def _enc_body(x_ref, we_ref, be_ref, w1_ref, dv_ref, o_ref):
    h = jnp.tanh(
        jnp.dot(x_ref[...], we_ref[...], preferred_element_type=jnp.float32)
        + be_ref[...])
    o_ref[...] = jnp.dot(
        h, w1_ref[...], preferred_element_type=jnp.float32) * dv_ref[...]


def _mid_body(p0_ref, p1_ref, b1_ref, w2_ref, dv_ref, o_ref):
    h = jnp.tanh((p0_ref[...] + p1_ref[...]) * dv_ref[...] + b1_ref[...])
    o_ref[...] = jnp.dot(
        h, w2_ref[...], preferred_element_type=jnp.float32) * dv_ref[...]


def _out_body(p0_ref, p1_ref, b2_ref, dv_ref, o_ref):
    o_ref[...] = jnp.tanh(
        (p0_ref[...] + p1_ref[...]) * dv_ref[...] + b2_ref[...])


_RB = 1000
_GRID = N // _RB


def kernel(x, edge_index, edge_attr, We, be, W1, b1, W2, b2):
    src = edge_index[0].astype(jnp.int32)
    dst = edge_index[1].astype(jnp.int32)
    ew = edge_attr.astype(jnp.float32)
    nodes = jnp.arange(NPAD, dtype=jnp.int32)
    pad_i = jnp.arange(EEXT - E - NPAD, dtype=jnp.int32)
    src_ext = jnp.concatenate(
        [src, jnp.minimum(nodes, N - 1), pad_i % N])
    dst_ext = jnp.concatenate([dst, nodes, N + pad_i % (NPAD - N)])
    ew_ext = jnp.concatenate(
        [ew, (nodes < N).astype(jnp.float32),
         jnp.zeros(EEXT - E - NPAD, jnp.float32)])
    dst2d = dst_ext.reshape(NW, CPW, CH)
    ew2d = ew_ext.reshape(NW, CPW, CH)
    ed = jnp.concatenate([
        src_ext.reshape(NW, CPW, 1, CH),
        dst_ext.reshape(NW, CPW, 1, CH),
    ], axis=2)
    be2 = be.reshape(1, D)
    b12 = b1.reshape(1, D)
    b22 = b2.reshape(1, D)

    deg_parts = _deg_call(dst2d, ew2d)

    dinv2d = pl.pallas_call(
        _dinv_body,
        out_shape=jax.ShapeDtypeStruct((NPAD // 128, 128), jnp.float32),
    )(deg_parts.reshape(NC, NPAD // 128, 128))
    dinv_col = dinv2d.reshape(NPAD, 1)[:N]

    blk_row = pl.BlockSpec((_RB, D), lambda i: (i, 0))
    blk_mat = pl.BlockSpec((D, D), lambda i: (0, 0))
    blk_vec = pl.BlockSpec((1, D), lambda i: (0, 0))
    blk_col = pl.BlockSpec((_RB, 1), lambda i: (i, 0))
    out_row = jax.ShapeDtypeStruct((N, D), jnp.float32)

    xw1 = pl.pallas_call(
        _enc_body,
        grid=(_GRID,),
        in_specs=[blk_row, blk_mat, blk_vec, blk_mat, blk_col],
        out_specs=blk_row,
        out_shape=out_row,
    )(x, We, be2, W1, dinv_col)

    parts1 = _spmm_call(xw1, ed, ew2d)

    xw2 = pl.pallas_call(
        _mid_body,
        grid=(_GRID,),
        in_specs=[blk_row, blk_row, blk_vec, blk_mat, blk_col],
        out_specs=blk_row,
        out_shape=out_row,
    )(parts1[0], parts1[1], b12, W2, dinv_col)

    parts2 = _spmm_call(xw2, ed, ew2d)

    out = pl.pallas_call(
        _out_body,
        grid=(_GRID,),
        in_specs=[blk_row, blk_row, blk_vec, blk_col],
        out_specs=blk_row,
        out_shape=out_row,
    )(parts2[0], parts2[1], b22, dinv_col)
    return out
